# baseline (speedup 1.0000x reference)
import functools

import jax
import jax.numpy as jnp
from jax import lax
from jax.experimental import pallas as pl
from jax.experimental.pallas import tpu as pltpu

_ALPHA = 1.0
_K = 5
_RADIUS = 0.07
_H2 = 0.03 * 0.03
_EPS = 1e-12
_INF = float("inf")


def _loss_kernel(q_ref, c2_ref, o_ref, *, n, blk_r):
    q = q_ref[0]
    c2 = c2_ref[0]
    dot2 = lax.dot_general(q, c2, (((1,), (0,)), ((), ())),
                           precision=lax.Precision.DEFAULT)
    qx, qy, qz = q[:, 0:1], q[:, 1:2], q[:, 2:3]
    cx, cy, cz = c2[0:1, :] * 0.5, c2[1:2, :] * 0.5, c2[2:3, :] * 0.5
    sq_q = (qx * qx + qy * qy) + qz * qz
    sq_c = (cx * cx + cy * cy) + cz * cz
    s = sq_q + sq_c
    d2 = s - dot2
    q2x, q2y, q2z = 2.0 * qx, 2.0 * qy, 2.0 * qz
    ex = s - ((q2x * cx + q2y * cy) + q2z * cz)
    terms = jnp.zeros((blk_r,), jnp.float32)
    for p in range(_K):
        m = jnp.min(d2, axis=1, keepdims=True)
        hit = d2 == m
        if p > 0:
            ev = jnp.sum(jnp.where(hit, ex, 0.0), axis=1)
            d2c = jnp.maximum(ev, _EPS)
            dist = jnp.sqrt(d2c)
            w = jnp.exp(-d2c / _H2)
            terms = terms + (_RADIUS - dist) * w
        if p < _K - 1:
            d2 = jnp.where(hit, _INF, d2)
    o_ref[pl.program_id(0), pl.program_id(1)] = jnp.sum(terms)


def kernel(array1):
    pred = array1
    b, n, _ = pred.shape
    blk_r = 1024
    qp = jnp.pad(pred, ((0, 0), (0, 0), (0, 125)))
    cp2 = jnp.transpose(2.0 * qp, (0, 2, 1))
    grid = (b, n // blk_r)
    partials = pl.pallas_call(
        functools.partial(_loss_kernel, n=n, blk_r=blk_r),
        grid=grid,
        in_specs=[
            pl.BlockSpec((1, blk_r, 128), lambda i, r: (i, r, 0)),
            pl.BlockSpec((1, 128, n), lambda i, r: (i, 0, 0)),
        ],
        out_specs=pl.BlockSpec(grid, lambda i, r: (0, 0),
                               memory_space=pltpu.SMEM),
        out_shape=jax.ShapeDtypeStruct(grid, jnp.float32),
    )(qp, cp2)
    total = jnp.sum(partials)
    return _ALPHA * (total / float(b * n * (_K - 1)))


# speedup vs baseline: 1.0484x; 1.0484x over previous
import functools

import jax
import jax.numpy as jnp
from jax import lax
from jax.experimental import pallas as pl
from jax.experimental.pallas import tpu as pltpu

_ALPHA = 1.0
_K = 5
_RADIUS = 0.07
_H2 = 0.03 * 0.03
_EPS = 1e-12
_INF = float("inf")


def _loss_kernel(q_ref, c2_ref, o_ref, *, n, blk_r):
    q = q_ref[0]
    c2 = c2_ref[0]
    dot2 = lax.dot_general(q, c2, (((1,), (0,)), ((), ())),
                           precision=lax.Precision.DEFAULT)
    qx, qy, qz = q[:, 0:1], q[:, 1:2], q[:, 2:3]
    cx, cy, cz = c2[0:1, :] * 0.5, c2[1:2, :] * 0.5, c2[2:3, :] * 0.5
    sq_q = (qx * qx + qy * qy) + qz * qz
    sq_c = (cx * cx + cy * cy) + cz * cz
    s = sq_q + sq_c
    d2 = s - dot2
    q2x, q2y, q2z = 2.0 * qx, 2.0 * qy, 2.0 * qz
    ex = s - ((q2x * cx + q2y * cy) + q2z * cz)
    terms = jnp.zeros((blk_r,), jnp.float32)
    for p in range(_K):
        m = jnp.min(d2, axis=1, keepdims=True)
        hit = d2 == m
        if p > 0:
            ev = jnp.sum(jnp.where(hit, ex, 0.0), axis=1)
            d2c = jnp.maximum(ev, _EPS)
            dist = jnp.sqrt(d2c)
            w = jnp.exp(-d2c / _H2)
            terms = terms + (_RADIUS - dist) * w
        if p < _K - 1:
            d2 = jnp.where(hit, _INF, d2)
    o_ref[pl.program_id(0), pl.program_id(1)] = jnp.sum(terms)


def kernel(array1):
    pred = array1
    b, n, _ = pred.shape
    blk_r = 1024
    cp2 = jnp.transpose(2.0 * pred, (0, 2, 1))
    grid = (b, n // blk_r)
    partials = pl.pallas_call(
        functools.partial(_loss_kernel, n=n, blk_r=blk_r),
        grid=grid,
        in_specs=[
            pl.BlockSpec((1, blk_r, 3), lambda i, r: (i, r, 0)),
            pl.BlockSpec((1, 3, n), lambda i, r: (i, 0, 0)),
        ],
        out_specs=pl.BlockSpec(grid, lambda i, r: (0, 0),
                               memory_space=pltpu.SMEM),
        out_shape=jax.ShapeDtypeStruct(grid, jnp.float32),
    )(pred, cp2)
    total = jnp.sum(partials)
    return _ALPHA * (total / float(b * n * (_K - 1)))
